# Initial kernel scaffold; baseline (speedup 1.0000x reference)
import jax
import jax.numpy as jnp
from jax.experimental import pallas as pl


def kernel(x, edge_index, batch, params):
    raise NotImplementedError("write your pallas kernel here")


# speedup vs baseline: 3.3349x; 3.3349x over previous
import functools

import jax
import jax.numpy as jnp
from jax import lax
from jax.experimental import pallas as pl
from jax.experimental.pallas import tpu as pltpu
from jax.experimental.pallas import tpu_sc as plsc

F32 = jnp.float32
I32 = jnp.int32

N = 10000
NPAD = 10240
NG = 64
NC, NS = 2, 16
NW = NC * NS
B = 128
E_REAL = 160000
EP = E_REAL + N
EPW = 5504
NBLK = EPW // B
EPAD = EPW * NW
ZROWS = NPAD // NS


def _mesh():
    return plsc.VectorSubcoreMesh(core_axis_name="c", subcore_axis_name="s",
                                  num_cores=NC, num_subcores=NS)


@functools.cache
def _sc_spmm(width):
    @functools.partial(
        pl.kernel,
        out_type=jax.ShapeDtypeStruct((NC, NPAD, width), F32),
        mesh=_mesh(),
        compiler_params=pltpu.CompilerParams(needs_layout_passes=False),
        scratch_types=[
            pltpu.VMEM((B,), I32),
            pltpu.VMEM((B,), I32),
            pltpu.VMEM((B, width), F32),
            pltpu.VMEM_SHARED((NPAD, width), F32),
            pltpu.SemaphoreType.DMA,
        ],
    )
    def k(xw_hbm, src_hbm, dst_hbm, zeros_hbm, out_hbm,
          src_v, dst_v, rows_v, acc_sh, sem):
        cid = lax.axis_index("c")
        sid = lax.axis_index("s")
        wid = sid * NC + cid
        pltpu.sync_copy(zeros_hbm, acc_sh.at[pl.ds(sid * ZROWS, ZROWS)])
        plsc.subcore_barrier()

        @pl.loop(0, NBLK)
        def _(i):
            base = wid * EPW + i * B
            pltpu.sync_copy(src_hbm.at[pl.ds(base, B)], src_v)
            pltpu.sync_copy(dst_hbm.at[pl.ds(base, B)], dst_v)
            pltpu.async_copy(xw_hbm.at[src_v], rows_v, sem).wait()
            pltpu.sync_copy(rows_v, acc_sh.at[dst_v], add=True)

        plsc.subcore_barrier()
        pltpu.sync_copy(acc_sh.at[pl.ds(sid * ZROWS, ZROWS)],
                        out_hbm.at[cid, pl.ds(sid * ZROWS, ZROWS)])

    return k


@functools.cache
def _sc_spmm_scaled(width):
    @functools.partial(
        pl.kernel,
        out_type=jax.ShapeDtypeStruct((NC, NPAD, width), F32),
        mesh=_mesh(),
        compiler_params=pltpu.CompilerParams(needs_layout_passes=False),
        scratch_types=[
            pltpu.VMEM((B,), I32),
            pltpu.VMEM((B,), I32),
            pltpu.VMEM((B,), F32),
            pltpu.VMEM((B, width), F32),
            pltpu.VMEM_SHARED((NPAD, width), F32),
            pltpu.SemaphoreType.DMA,
        ],
    )
    def k(xw_hbm, src_hbm, dst_hbm, ew_hbm, zeros_hbm, out_hbm,
          src_v, dst_v, e_v, rows_v, acc_sh, sem):
        cid = lax.axis_index("c")
        sid = lax.axis_index("s")
        wid = sid * NC + cid
        pltpu.sync_copy(zeros_hbm, acc_sh.at[pl.ds(sid * ZROWS, ZROWS)])
        plsc.subcore_barrier()

        @pl.loop(0, NBLK)
        def _(i):
            base = wid * EPW + i * B
            pltpu.sync_copy(src_hbm.at[pl.ds(base, B)], src_v)
            pltpu.sync_copy(dst_hbm.at[pl.ds(base, B)], dst_v)
            pltpu.sync_copy(ew_hbm.at[pl.ds(base, B)], e_v)
            pltpu.async_copy(xw_hbm.at[src_v], rows_v, sem).wait()

            @pl.loop(0, B)
            def _(b):
                ebc = plsc.load_gather(e_v, [jnp.zeros((16,), I32) + b])

                @pl.loop(0, width // 16)
                def _(j):
                    rows_v[b, pl.ds(j * 16, 16)] = (
                        rows_v[b, pl.ds(j * 16, 16)] * ebc)

            pltpu.sync_copy(rows_v, acc_sh.at[dst_v], add=True)

        plsc.subcore_barrier()
        pltpu.sync_copy(acc_sh.at[pl.ds(sid * ZROWS, ZROWS)],
                        out_hbm.at[cid, pl.ds(sid * ZROWS, ZROWS)])

    return k


@functools.cache
def _sc_edge_exp():
    @functools.partial(
        pl.kernel,
        out_type=jax.ShapeDtypeStruct((EPAD,), F32),
        mesh=_mesh(),
        compiler_params=pltpu.CompilerParams(needs_layout_passes=False),
        scratch_types=[
            pltpu.VMEM((NPAD,), F32),
            pltpu.VMEM((NPAD,), F32),
            pltpu.VMEM((B,), I32),
            pltpu.VMEM((B,), I32),
            pltpu.VMEM((B,), F32),
        ],
    )
    def k(s_hbm, t_hbm, src_hbm, dst_hbm, e_hbm,
          s_v, t_v, src_v, dst_v, e_v):
        cid = lax.axis_index("c")
        sid = lax.axis_index("s")
        wid = sid * NC + cid
        pltpu.sync_copy(s_hbm, s_v)
        pltpu.sync_copy(t_hbm, t_v)

        @pl.loop(0, NBLK)
        def _(i):
            base = wid * EPW + i * B
            pltpu.sync_copy(src_hbm.at[pl.ds(base, B)], src_v)
            pltpu.sync_copy(dst_hbm.at[pl.ds(base, B)], dst_v)

            @pl.loop(0, B // 16)
            def _(j):
                si = src_v[pl.ds(j * 16, 16)]
                di = dst_v[pl.ds(j * 16, 16)]
                a = plsc.load_gather(s_v, [si]) + plsc.load_gather(t_v, [di])
                a = jnp.where(a >= 0.0, a, 0.2 * a)
                e_v[pl.ds(j * 16, 16)] = jnp.exp(a)

            pltpu.sync_copy(e_v, e_hbm.at[pl.ds(base, B)])

    return k


@functools.cache
def _sc_edge_norm():
    @functools.partial(
        pl.kernel,
        out_type=jax.ShapeDtypeStruct((EPAD,), F32),
        mesh=_mesh(),
        compiler_params=pltpu.CompilerParams(needs_layout_passes=False),
        scratch_types=[
            pltpu.VMEM((NPAD,), F32),
            pltpu.VMEM((B,), I32),
            pltpu.VMEM((B,), I32),
            pltpu.VMEM((B,), F32),
        ],
    )
    def k(d_hbm, src_hbm, dst_hbm, n_hbm, d_v, src_v, dst_v, n_v):
        cid = lax.axis_index("c")
        sid = lax.axis_index("s")
        wid = sid * NC + cid
        pltpu.sync_copy(d_hbm, d_v)

        @pl.loop(0, NBLK)
        def _(i):
            base = wid * EPW + i * B
            pltpu.sync_copy(src_hbm.at[pl.ds(base, B)], src_v)
            pltpu.sync_copy(dst_hbm.at[pl.ds(base, B)], dst_v)

            @pl.loop(0, B // 16)
            def _(j):
                si = src_v[pl.ds(j * 16, 16)]
                di = dst_v[pl.ds(j * 16, 16)]
                n_v[pl.ds(j * 16, 16)] = (plsc.load_gather(d_v, [si])
                                          * plsc.load_gather(d_v, [di]))

            pltpu.sync_copy(n_v, n_hbm.at[pl.ds(base, B)])

    return k


@functools.cache
def _sc_edge_coef():
    @functools.partial(
        pl.kernel,
        out_type=jax.ShapeDtypeStruct((EPAD,), F32),
        mesh=_mesh(),
        compiler_params=pltpu.CompilerParams(needs_layout_passes=False),
        scratch_types=[
            pltpu.VMEM((NPAD,), F32),
            pltpu.VMEM((B,), I32),
            pltpu.VMEM((B,), F32),
            pltpu.VMEM((B,), F32),
        ],
    )
    def k(den_hbm, e_hbm, dst_hbm, c_hbm, den_v, dst_v, e_v, c_v):
        cid = lax.axis_index("c")
        sid = lax.axis_index("s")
        wid = sid * NC + cid
        pltpu.sync_copy(den_hbm, den_v)

        @pl.loop(0, NBLK)
        def _(i):
            base = wid * EPW + i * B
            pltpu.sync_copy(dst_hbm.at[pl.ds(base, B)], dst_v)
            pltpu.sync_copy(e_hbm.at[pl.ds(base, B)], e_v)

            @pl.loop(0, B // 16)
            def _(j):
                di = dst_v[pl.ds(j * 16, 16)]
                e16 = e_v[pl.ds(j * 16, 16)]
                c_v[pl.ds(j * 16, 16)] = e16 / (plsc.load_gather(den_v, [di])
                                                + 1e-16)

            pltpu.sync_copy(c_v, c_hbm.at[pl.ds(base, B)])

    return k


def _bn_relu(y, g, bt):
    m = jnp.mean(y, axis=0, keepdims=True)
    yc = y - m
    v = jnp.mean(yc * yc, axis=0, keepdims=True)
    return jnp.maximum(yc / jnp.sqrt(v + 1e-5) * g + bt, 0.0)


_EB = 4096


def _tc_seg_onehot(dst_cols, w_cols):
    def body(dst_ref, w_ref, out_ref):
        d = dst_ref[...]
        hi = lax.shift_right_logical(d, 7)
        lo = d & 127
        a = (hi == lax.broadcasted_iota(I32, (1, NPAD // 128), 1)).astype(F32)
        bm = (lo == lax.broadcasted_iota(I32, (1, 128), 1)).astype(F32)
        part = lax.dot_general(a * w_ref[...], bm, (((0,), (0,)), ((), ())),
                               precision=lax.Precision.HIGHEST,
                               preferred_element_type=F32)

        @pl.when(pl.program_id(0) == 0)
        def _():
            out_ref[...] = jnp.zeros_like(out_ref)

        out_ref[...] += part

    return pl.pallas_call(
        body,
        grid=(EPAD // _EB,),
        in_specs=[pl.BlockSpec((_EB, 1), lambda i: (i, 0)),
                  pl.BlockSpec((_EB, 1), lambda i: (i, 0))],
        out_specs=pl.BlockSpec((NPAD // 128, 128), lambda i: (0, 0)),
        out_shape=jax.ShapeDtypeStruct((NPAD // 128, 128), F32),
    )(dst_cols, w_cols)


def _tc_dis(deg):
    def body(deg_ref, out_ref):
        d = deg_ref[...]
        rid = lax.broadcasted_iota(I32, (NPAD, 1), 0)
        ok = (d > 0.0) & (rid < N)
        out_ref[...] = jnp.where(ok, lax.rsqrt(jnp.maximum(d, 1e-12)), 0.0)

    return pl.pallas_call(
        body, out_shape=jax.ShapeDtypeStruct((NPAD, 1), F32))(deg)


def _tc_mm_scale(h, wt, dis, chunks):
    def body(h_ref, wt_ref, dis_ref, *out_refs):
        xw = jnp.dot(h_ref[...], wt_ref[...], preferred_element_type=F32)
        sc = dis_ref[0:N]
        for (c0, pc, wc), o in zip(chunks, out_refs):
            val = xw[:, c0:c0 + pc] * sc
            if wc > pc:
                val = jnp.concatenate(
                    [val, jnp.zeros((N, wc - pc), F32)], axis=1)
            o[0:N] = val
            o[N:NPAD] = jnp.zeros((NPAD - N, wc), F32)

    outs = tuple(jax.ShapeDtypeStruct((NPAD, wc), F32) for _, _, wc in chunks)
    return pl.pallas_call(body, out_shape=outs)(h, wt, dis)


def _tc_post(parts, chunks, b, g, bt, dout):
    nch = len(parts)

    def body(*refs):
        ps = refs[:nch]
        b_ref, g_ref, bt_ref, out_ref = refs[nch:]
        agg = jnp.concatenate(
            [p[0, 0:N, 0:pc] + p[1, 0:N, 0:pc]
             for p, (_, pc, _) in zip(ps, chunks)], axis=1)
        y = agg + b_ref[...]
        out_ref[...] = _bn_relu(y, g_ref[...], bt_ref[...])

    return pl.pallas_call(
        body, out_shape=jax.ShapeDtypeStruct((N, dout), F32))(
            *parts, b, g, bt)


def _tc_gat_pre(hs, wt, a_s, a_d, chunks):
    nh = len(hs)

    def body(*refs):
        h_refs = refs[:nh]
        wt_ref, as_ref, ad_ref = refs[nh:nh + 3]
        outs = refs[nh + 3:]
        chunk_outs, s_out, t_out = outs[:-2], outs[-2], outs[-1]
        xw = None
        r0 = 0
        for hr in h_refs:
            din_i = hr.shape[1]
            part = jnp.dot(hr[...], wt_ref[r0:r0 + din_i],
                           preferred_element_type=F32)
            xw = part if xw is None else xw + part
            r0 += din_i
        s = jnp.dot(xw, as_ref[...], preferred_element_type=F32)
        t = jnp.dot(xw, ad_ref[...], preferred_element_type=F32)
        for (c0, pc, wc), o in zip(chunks, chunk_outs):
            val = xw[:, c0:c0 + pc]
            if wc > pc:
                val = jnp.concatenate(
                    [val, jnp.zeros((N, wc - pc), F32)], axis=1)
            o[0:N] = val
            o[N:NPAD] = jnp.zeros((NPAD - N, wc), F32)
        s_out[0:N] = s
        s_out[N:NPAD] = jnp.zeros((NPAD - N, 1), F32)
        t_out[0:N] = t
        t_out[N:NPAD] = jnp.zeros((NPAD - N, 1), F32)

    outs = tuple(jax.ShapeDtypeStruct((NPAD, wc), F32) for _, _, wc in chunks)
    outs = outs + (jax.ShapeDtypeStruct((NPAD, 1), F32),
                   jax.ShapeDtypeStruct((NPAD, 1), F32))
    return pl.pallas_call(body, out_shape=outs)(*hs, wt, a_s, a_d)


def _tc_pool_mlp(xa, batch, p):
    def body(xa_ref, b_ref, l0w, l0b, g0, t0, l1w, l1b, g1, t1,
             l2w, l2b, g2, t2, l3w, l3b, g3, t3, ow, ob, out_ref):
        gid = lax.broadcasted_iota(I32, (1, NG), 1)
        mask = (b_ref[...] == gid).astype(F32)
        sums = lax.dot_general(mask, xa_ref[...],
                               (((0,), (0,)), ((), ())),
                               precision=lax.Precision.HIGHEST,
                               preferred_element_type=F32)
        cnt = jnp.sum(mask, axis=0)[:, None]
        h = sums / jnp.maximum(cnt, 1.0)
        for lw, lb, gg, tt in ((l0w, l0b, g0, t0), (l1w, l1b, g1, t1),
                               (l2w, l2b, g2, t2), (l3w, l3b, g3, t3)):
            y = jnp.dot(h, lw[...], preferred_element_type=F32) + lb[...]
            h = _bn_relu(y, gg[...], tt[...])
        out_ref[...] = jnp.dot(h, ow[...], preferred_element_type=F32) + ob[...]

    args = [xa, batch]
    for i in range(4):
        args += [p[f'lin{i}_w'].T, p[f'lin{i}_b'][None, :],
                 p[f'bn_lin{i}_g'][None, :], p[f'bn_lin{i}_b'][None, :]]
    args += [p['out_w'].T, p['out_b'][None, :]]
    return pl.pallas_call(
        body, out_shape=jax.ShapeDtypeStruct((NG, 1), F32))(*args)


def _chunks(dout):
    if dout == 256:
        return ((0, 128, 128), (128, 128, 128))
    if dout == 128:
        return ((0, 128, 128),)
    return ((0, 64, 128),)


def kernel(x, edge_index, batch, params):
    p = params
    pad_i = jnp.full((EPAD - EP,), NPAD - 1, I32)
    loop = jnp.arange(N, dtype=I32)
    src = jnp.concatenate([edge_index[0].astype(I32), loop, pad_i])
    dst = jnp.concatenate([edge_index[1].astype(I32), loop, pad_i])

    zeros_by_w = {}

    def zw(width):
        if width not in zeros_by_w:
            zeros_by_w[width] = jnp.zeros((ZROWS, width), F32)
        return zeros_by_w[width]

    dst_cols = dst[:, None]
    ones_cols = jnp.ones((EPAD, 1), F32)
    ones_col_n = jnp.ones((NPAD, 1), F32)
    deg = _tc_seg_onehot(dst_cols, ones_cols).reshape(NPAD, 1)
    dis = _tc_dis(deg)
    norm = _sc_edge_norm()(dis.reshape(NPAD), src, dst)

    def gcn(h, wk, bnk):
        w = p[wk + '_w']
        dout = w.shape[0]
        chunks = _chunks(dout)
        xwcs = _tc_mm_scale(h, w.T, ones_col_n, chunks)
        parts = [_sc_spmm_scaled(wc)(xwc, src, dst, norm, zw(wc))
                 for (_, _, wc), xwc in zip(chunks, xwcs)]
        return _tc_post(parts, chunks, p[wk + '_b'][None, :],
                        p[bnk + '_g'][None, :], p[bnk + '_b'][None, :], dout)

    def gat(hs, wk, bnk):
        w = p[wk + '_w']
        dout = w.shape[0]
        chunks = _chunks(dout)
        outs = _tc_gat_pre(hs, w.T, p[wk + '_as'][:, None],
                           p[wk + '_ad'][:, None], chunks)
        xwcs, s2, t2 = outs[:-2], outs[-2], outs[-1]
        e = _sc_edge_exp()(s2.reshape(NPAD), t2.reshape(NPAD), src, dst)
        den = _tc_seg_onehot(dst_cols, e[:, None]).reshape(NPAD)
        coef = _sc_edge_coef()(den, e, dst)
        parts = [_sc_spmm_scaled(wc)(xwc, src, dst, coef, zw(wc))
                 for (_, _, wc), xwc in zip(chunks, xwcs)]
        return _tc_post(parts, chunks, p[wk + '_b'][None, :],
                        p[bnk + '_g'][None, :], p[bnk + '_b'][None, :], dout)

    xl = gcn(x, 'gcn_l1', 'bn_l1')
    xl = gat([xl], 'gat_l1', 'bn_gl1')
    xm = gcn(x, 'gcn_m1', 'bn_m1')
    xm = gcn(xm, 'gcn_m2', 'bn_m2')
    xm = gat([xm], 'gat_m2', 'bn_gm2')
    xr = gcn(x, 'gcn_r1', 'bn_r1')
    xr = gcn(xr, 'gcn_r2', 'bn_r2')
    xr = gcn(xr, 'gcn_r3', 'bn_r3')
    xr = gat([xr], 'gat_r3', 'bn_gr3')
    xa = gat([xl, xm, xr], 'gat_all', 'bn_all')
    return _tc_pool_mlp(xa, batch[:, None], p)
